# baseline (speedup 1.0000x reference)
import functools

import jax
import jax.numpy as jnp
from jax import lax
from jax.experimental import pallas as pl
from jax.experimental.pallas import tpu as pltpu
from jax.experimental.pallas import tpu_sc as plsc

GROUP = 5
RCH_G = 1
RCH = 8
SLAB = 8
CPS = SLAB // RCH_G
STRIDE = 8
LANES = 16


def _sc_gather_pool(table, idx_flat, ng, d, nw):
    mesh = plsc.VectorSubcoreMesh(core_axis_name="c", subcore_axis_name="s")
    n_slabs = ng // SLAB
    base = n_slabs // nw
    extra = n_slabs - base * nw
    max_slabs = base + (1 if extra else 0)
    win = max_slabs * CPS * STRIDE
    nvec = d // LANES

    @functools.partial(
        pl.kernel,
        mesh=mesh,
        out_type=jax.ShapeDtypeStruct((ng * d,), jnp.float32),
        scratch_types=[
            pltpu.VMEM((win,), jnp.int32),
            pltpu.VMEM((RCH, d), jnp.float32),
            pltpu.VMEM((SLAB * d,), jnp.float32),
            pltpu.SemaphoreType.DMA,
        ],
    )
    def pool_kernel(table_hbm, idx_hbm, out_hbm, idx_v, rows_v, slab_v, sem):
        wid = lax.axis_index("s") * 2 + lax.axis_index("c")
        start = base * wid + jnp.minimum(wid, extra)
        my_slabs = base + jnp.where(wid < extra, 1, 0)
        pltpu.sync_copy(idx_hbm.at[pl.ds(start * (CPS * STRIDE), win)], idx_v)

        def do_slab(s, carry):
            for cs in range(CPS):
                c = s * CPS + cs
                pltpu.async_copy(
                    table_hbm.at[idx_v.at[pl.ds(c * STRIDE, RCH)]],
                    rows_v,
                    sem,
                ).wait()

                def vbody(i, _, cs=cs):
                    o = i * LANES
                    acc = rows_v[0, pl.ds(o, LANES)]
                    for j in range(1, GROUP):
                        acc = acc + rows_v[j, pl.ds(o, LANES)]
                    slab_v[pl.ds(cs * d + o, LANES)] = acc * (1.0 / GROUP)
                    return _

                lax.fori_loop(0, nvec, vbody, 0)

            pltpu.sync_copy(
                slab_v, out_hbm.at[pl.ds((start + s) * (SLAB * d), SLAB * d)]
            )
            return carry

        lax.fori_loop(0, my_slabs, do_slab, 0)

    return pool_kernel(table, idx_flat)


def _tc_matmul(pooled, w_bf16, ng, d, bm, bk):
    def body(a_ref, w_ref, o_ref):
        k = pl.program_id(1)
        a = a_ref[...].astype(jnp.bfloat16)
        part = lax.dot_general(
            a,
            w_ref[...],
            (((1,), (1,)), ((), ())),
            preferred_element_type=jnp.float32,
        )

        @pl.when(k == 0)
        def _():
            o_ref[...] = part

        @pl.when(k != 0)
        def _():
            o_ref[...] += part

    return pl.pallas_call(
        body,
        grid=(ng // bm, d // bk),
        in_specs=[
            pl.BlockSpec((bm, bk), lambda i, k: (i, k)),
            pl.BlockSpec((d, bk), lambda i, k: (0, k)),
        ],
        out_specs=pl.BlockSpec((bm, d), lambda i, k: (i, 0)),
        out_shape=jax.ShapeDtypeStruct((ng, d), jnp.float32),
    )(pooled, w_bf16)


def kernel(audio_ids, embed_table, W_out):
    b, s = audio_ids.shape
    v, d = embed_table.shape
    ng = (b * s) // GROUP
    nw = 32

    ids = audio_ids.reshape(-1).astype(jnp.int32)
    n_slabs = ng // SLAB
    max_slabs = n_slabs // nw + (1 if n_slabs % nw else 0)
    n_chunks = ng // RCH_G
    pad_chunks = nw * max_slabs * CPS - n_chunks
    idx_2d = ids.reshape(n_chunks, RCH_G * GROUP)
    idx_2d = jnp.pad(idx_2d, ((0, pad_chunks), (0, STRIDE - RCH_G * GROUP)))
    idx_flat = idx_2d.reshape(-1)

    pooled = _sc_gather_pool(embed_table, idx_flat, ng, d, nw).reshape(ng, d)
    out = _tc_matmul(pooled, W_out.astype(jnp.bfloat16), ng, d, bm=800, bk=512)
    return out.reshape(b, s // GROUP, d)


# speedup vs baseline: 3.0704x; 3.0704x over previous
import functools

import jax
import jax.numpy as jnp
from jax import lax
from jax.experimental import pallas as pl
from jax.experimental.pallas import tpu as pltpu
from jax.experimental.pallas import tpu_sc as plsc

GROUP = 5


def _sc_gather(table, idx_flat, n_rows, d, nw, k_rows):
    mesh = plsc.VectorSubcoreMesh(core_axis_name="c", subcore_axis_name="s")
    n_chunks = n_rows // k_rows
    base_chunks = n_chunks // nw
    extra = n_chunks - base_chunks * nw
    max_chunks = base_chunks + (1 if extra else 0)
    win = max_chunks * k_rows

    @functools.partial(
        pl.kernel,
        mesh=mesh,
        out_type=jax.ShapeDtypeStruct((n_rows, d), jnp.float32),
        scratch_types=[
            pltpu.VMEM((win,), jnp.int32),
            pltpu.VMEM((k_rows, d), jnp.float32),
            pltpu.VMEM((k_rows, d), jnp.float32),
            pltpu.SemaphoreType.DMA,
            pltpu.SemaphoreType.DMA,
        ],
    )
    def gather_kernel(
        table_hbm, idx_hbm, out_hbm, idx_v, rows0, rows1, sem0, sem1
    ):
        rows = (rows0, rows1)
        sems = (sem0, sem1)
        wid = lax.axis_index("s") * 2 + lax.axis_index("c")
        start = base_chunks * wid + jnp.minimum(wid, extra)
        my_chunks = base_chunks + jnp.where(wid < extra, 1, 0)
        pltpu.sync_copy(idx_hbm.at[pl.ds(start * k_rows, win)], idx_v)

        def fire(c, b):
            pltpu.async_copy(
                table_hbm.at[idx_v.at[pl.ds(c * k_rows, k_rows)]],
                rows[b],
                sems[b],
            )

        def wait(c, b):
            pltpu.make_async_copy(
                table_hbm.at[idx_v.at[pl.ds(c * k_rows, k_rows)]],
                rows[b],
                sems[b],
            ).wait()

        fire(0, 0)

        def body(p, carry):
            for b in range(2):
                c = 2 * p + b

                @pl.when(c < my_chunks)
                def _(c=c, b=b):
                    nxt = c + 1

                    @pl.when(nxt < my_chunks)
                    def _():
                        fire(nxt, 1 - b)

                    wait(c, b)
                    pltpu.sync_copy(
                        rows[b],
                        out_hbm.at[pl.ds((start + c) * k_rows, k_rows)],
                    )

            return carry

        lax.fori_loop(0, (max_chunks + 1) // 2, body, 0)

    return gather_kernel(table, idx_flat)


def _tc_pool_matmul(g3, w_bf16, ng, d, bm, bk):
    def body(a_ref, w_ref, o_ref):
        k = pl.program_id(1)
        s = a_ref[0] + a_ref[1] + a_ref[2] + a_ref[3] + a_ref[4]
        pooled = (s * (1.0 / GROUP)).astype(jnp.bfloat16)
        part = lax.dot_general(
            pooled,
            w_ref[...],
            (((1,), (1,)), ((), ())),
            preferred_element_type=jnp.float32,
        )

        @pl.when(k == 0)
        def _():
            o_ref[...] = part

        @pl.when(k != 0)
        def _():
            o_ref[...] += part

    return pl.pallas_call(
        body,
        grid=(ng // bm, d // bk),
        in_specs=[
            pl.BlockSpec((GROUP, bm, bk), lambda i, k: (0, i, k)),
            pl.BlockSpec((d, bk), lambda i, k: (0, k)),
        ],
        out_specs=pl.BlockSpec((bm, d), lambda i, k: (i, 0)),
        out_shape=jax.ShapeDtypeStruct((ng, d), jnp.float32),
    )(g3, w_bf16)


def kernel(audio_ids, embed_table, W_out):
    b, s = audio_ids.shape
    v, d = embed_table.shape
    ng = (b * s) // GROUP

    nw = 32
    k_rows = 16
    nph = 2
    gph = ng // nph
    rph = gph * GROUP

    ids = audio_ids.reshape(-1).astype(jnp.int32)
    w16 = W_out.astype(jnp.bfloat16)

    outs = []
    for h in range(nph):
        ids_h = ids[h * rph : (h + 1) * rph]
        idx_perm = ids_h.reshape(gph, GROUP).T.reshape(-1)
        n_chunks = rph // k_rows
        max_chunks = n_chunks // nw + (1 if n_chunks % nw else 0)
        pad = nw * max_chunks * k_rows - rph
        idx_perm = jnp.concatenate([idx_perm, jnp.zeros((pad,), jnp.int32)])
        gathered = _sc_gather(embed_table, idx_perm, rph, d, nw, k_rows)
        g3 = gathered.reshape(GROUP, gph, d)
        outs.append(_tc_pool_matmul(g3, w16, gph, d, bm=800, bk=512))

    out = jnp.concatenate(outs, axis=0)
    return out.reshape(b, s // GROUP, d)
